# Initial kernel scaffold; baseline (speedup 1.0000x reference)
import jax
import jax.numpy as jnp
from jax.experimental import pallas as pl


def kernel(x, Wc, bc, Wk, bk, Wv, bv, Wg, bg):
    raise NotImplementedError("write your pallas kernel here")


# speedup vs baseline: 447.0099x; 447.0099x over previous
import math

import jax
import jax.numpy as jnp
from jax.experimental import pallas as pl
from jax.experimental.pallas import tpu as pltpu

_B = 2
_S = 512
_L_CMP = 32
_L_SLC = 32
_L_WIN = 128
_DIM = 512
_H = 8
_STRIDE = 16
_TOPK = 2
_HD = _DIM // _H
_KV_CMP = (_S - _L_CMP) // _STRIDE + 1
_KV_SLC = _S // _L_SLC
_SCALE = 1.0 / math.sqrt(_HD)
_NEG_INF = float("-inf")


def _softmax(s):
    m = jnp.max(s, axis=-1, keepdims=True)
    e = jnp.exp(s - m)
    return e / jnp.sum(e, axis=-1, keepdims=True)


def _fused_kernel(x_ref, wc_ref, bc_ref, wk_ref, bk_ref, wv_ref, bv_ref,
                  wg_ref, bg_ref, out_ref, k_scr, v_scr):
    x = x_ref[0]
    qkv = jnp.dot(x, wc_ref[...].T, preferred_element_type=jnp.float32)
    qkv = qkv + bc_ref[...]
    q = qkv[:, :_DIM]
    k = qkv[:, _DIM:2 * _DIM]
    v = qkv[:, 2 * _DIM:]
    k_scr[...] = k
    v_scr[...] = v
    gate = jnp.dot(x, wg_ref[...].T, preferred_element_type=jnp.float32)
    gate = gate + bg_ref[...]

    r = jax.lax.broadcasted_iota(jnp.int32, (_KV_CMP, _S), 0)
    c = jax.lax.broadcasted_iota(jnp.int32, (_KV_CMP, _S), 1)
    off = c - r * _STRIDE
    mk = jnp.zeros((_KV_CMP, _S), dtype=jnp.float32)
    mv = jnp.zeros((_KV_CMP, _S), dtype=jnp.float32)
    for l in range(_L_CMP):
        sel = (off == l).astype(jnp.float32)
        mk = mk + sel * wk_ref[0, l]
        mv = mv + sel * wv_ref[0, l]
    k_cmp = jnp.dot(mk, k, preferred_element_type=jnp.float32) + bk_ref[0, 0]
    v_cmp = jnp.dot(mv, v, preferred_element_type=jnp.float32) + bv_ref[0, 0]

    ii_c = jax.lax.broadcasted_iota(jnp.int32, (_S, _KV_CMP), 0)
    jj_c = jax.lax.broadcasted_iota(jnp.int32, (_S, _KV_CMP), 1)
    cmp_valid = jj_c < ii_c
    ii_w = jax.lax.broadcasted_iota(jnp.int32, (_S, _S), 0)
    jj_w = jax.lax.broadcasted_iota(jnp.int32, (_S, _S), 1)
    win_valid = (jj_w <= ii_w) & (jj_w >= ii_w - _L_WIN)

    rr = jax.lax.broadcasted_iota(jnp.int32, (_KV_CMP, _KV_SLC), 0)
    jj_s = jax.lax.broadcasted_iota(jnp.int32, (_KV_CMP, _KV_SLC), 1)
    m_slc = ((rr == 2 * jj_s).astype(jnp.float32)
             + 2.0 * (rr == 2 * jj_s + 1).astype(jnp.float32)
             + (rr == 2 * jj_s + 2).astype(jnp.float32))

    lane16 = jax.lax.broadcasted_iota(jnp.int32, (1, _KV_SLC), 1)

    k_sum = jnp.zeros((_TOPK * _L_SLC, _HD), dtype=jnp.float32)
    v_sum = jnp.zeros((_TOPK * _L_SLC, _HD), dtype=jnp.float32)
    q_last = []

    g0 = gate[:, 0:1]
    g2 = gate[:, 2:3]

    for h in range(_H):
        c0 = h * _HD
        qh = q[:, c0:c0 + _HD]
        kh = k[:, c0:c0 + _HD]
        vh = v[:, c0:c0 + _HD]
        kch = k_cmp[:, c0:c0 + _HD]
        vch = v_cmp[:, c0:c0 + _HD]

        cs = jnp.dot(qh, kch.T, preferred_element_type=jnp.float32) * _SCALE
        cs = jnp.where(cmp_valid, cs, _NEG_INF)
        p_cmp = _softmax(cs)
        cmp_o = jnp.dot(p_cmp, vch, preferred_element_type=jnp.float32)

        ws = jnp.dot(qh, kh.T, preferred_element_type=jnp.float32)
        ws = jnp.where(win_valid, ws, _NEG_INF) * _SCALE
        p_win = _softmax(ws)
        win_o = jnp.dot(p_win, vh, preferred_element_type=jnp.float32)

        out_ref[0, :, c0:c0 + _HD] = g0 * cmp_o + g2 * win_o

        p_last = p_cmp[_S - 1:_S, :]
        p_slc = jnp.dot(p_last, m_slc, preferred_element_type=jnp.float32)
        m0 = jnp.max(p_slc)
        i0 = jnp.min(jnp.where(p_slc == m0, lane16, _KV_SLC))
        p_m = jnp.where(lane16 == i0, _NEG_INF, p_slc)
        m1 = jnp.max(p_m)
        i1 = jnp.min(jnp.where(p_m == m1, lane16, _KV_SLC))
        ka = k_scr[pl.ds(i0 * _L_SLC, _L_SLC), c0:c0 + _HD]
        kb = k_scr[pl.ds(i1 * _L_SLC, _L_SLC), c0:c0 + _HD]
        va = v_scr[pl.ds(i0 * _L_SLC, _L_SLC), c0:c0 + _HD]
        vb = v_scr[pl.ds(i1 * _L_SLC, _L_SLC), c0:c0 + _HD]
        k_sum = k_sum + jnp.concatenate([ka, kb], axis=0)
        v_sum = v_sum + jnp.concatenate([va, vb], axis=0)
        q_last.append(qh[_S - 1:_S, :])

    g1_15 = gate[_KV_SLC - 1, 1]
    for h in range(_H):
        c0 = h * _HD
        a = jnp.dot(q_last[h], k_sum.T, preferred_element_type=jnp.float32)
        p = _softmax(a * _SCALE)
        o = jnp.dot(p, v_sum, preferred_element_type=jnp.float32)
        cur = out_ref[0, _KV_SLC - 1:_KV_SLC, c0:c0 + _HD]
        out_ref[0, _KV_SLC - 1:_KV_SLC, c0:c0 + _HD] = cur + g1_15 * o


def kernel(x, Wc, bc, Wk, bk, Wv, bv, Wg, bg):
    bc2 = bc.reshape(1, 3 * _DIM)
    bk2 = bk.reshape(1, 1)
    bv2 = bv.reshape(1, 1)
    bg2 = bg.reshape(1, 3)

    def full(shape):
        return pl.BlockSpec(shape, lambda b: (0,) * len(shape))

    return pl.pallas_call(
        _fused_kernel,
        grid=(_B,),
        in_specs=[
            pl.BlockSpec((1, _S, _DIM), lambda b: (b, 0, 0)),
            full((3 * _DIM, _DIM)),
            full((1, 3 * _DIM)),
            full((1, _L_CMP)),
            full((1, 1)),
            full((1, _L_CMP)),
            full((1, 1)),
            full((3, _DIM)),
            full((1, 3)),
        ],
        out_specs=pl.BlockSpec((1, _S, _DIM), lambda b: (b, 0, 0)),
        out_shape=jax.ShapeDtypeStruct((_B, _S, _DIM), jnp.float32),
        scratch_shapes=[
            pltpu.VMEM((_S, _DIM), jnp.float32),
            pltpu.VMEM((_S, _DIM), jnp.float32),
        ],
        compiler_params=pltpu.CompilerParams(
            dimension_semantics=("parallel",),
        ),
    )(x, Wc, bc2, Wk, bk2, Wv, bv2, Wg, bg2)
